# baseline (speedup 1.0000x reference)
import functools

import jax
import jax.numpy as jnp
from jax import lax
from jax.experimental import pallas as pl
from jax.experimental.pallas import tpu as pltpu
from jax.experimental.pallas import tpu_sc as plsc

N_NODES = 10000
N_EDGES = 320000
D = 128

NC = 2
NS = 16
HALF = 5120
TRASH = HALF
AROWS = HALF + 8
CH = 128
HCH = 80
NCH = 2 * HCH
EPS = NCH * CH
EPAD = NS * EPS
RPS = HALF // NS


def _sc_agg_body(src_hbm, dst_hbm, x_hbm, out_hbm,
                 srcv, dstv, r0, r1, r2, r3, zbuf, aggsh,
                 gs0, gs1, gs2, gs3, ss0, ss1, ss2, ss3):
    rows = (r0, r1, r2, r3)
    gsem = (gs0, gs1, gs2, gs3)
    ssem = (ss0, ss1, ss2, ss3)
    cid = lax.axis_index("c")
    sid = lax.axis_index("s")
    lo = cid * HALF

    for r in range(8):
        for c in range(D // 16):
            zbuf[r, pl.ds(c * 16, 16)] = jnp.zeros((16,), jnp.float32)

    def _zfill(z, carry):
        pltpu.sync_copy(zbuf, aggsh.at[pl.ds(sid * RPS + z * 8, 8)])
        return carry
    lax.fori_loop(0, RPS // 8, _zfill, 0)

    @pl.when(sid == NS - 1)
    def _zero_trash():
        pltpu.sync_copy(zbuf, aggsh.at[pl.ds(HALF, 8)])

    plsc.subcore_barrier()

    def _remap(i):
        for j in range(CH // 16):
            t = dstv[i, pl.ds(j * 16, 16)] - lo
            oob = (t < 0) | (t >= HALF)
            dstv[i, pl.ds(j * 16, 16)] = jnp.where(oob, TRASH, t)

    def _step(j, bi, wait_scatter, prefetch):
        pltpu.make_async_copy(x_hbm.at[srcv.at[j]], rows[bi], gsem[bi]).wait()
        _remap(j)
        if wait_scatter:
            pb = (bi + 3) % 4
            pltpu.make_async_copy(
                rows[pb], aggsh.at[dstv.at[j - 1]], ssem[pb]).wait()
        pltpu.async_copy(rows[bi], aggsh.at[dstv.at[j]], ssem[bi], add=True)
        if prefetch:
            nb = (bi + 2) % 4
            pltpu.async_copy(x_hbm.at[srcv.at[j + 2]], rows[nb], gsem[nb])

    for h in range(2):
        pltpu.sync_copy(src_hbm.at[sid, h], srcv)
        pltpu.sync_copy(dst_hbm.at[sid, h], dstv)

        pltpu.async_copy(x_hbm.at[srcv.at[0]], r0, gs0)
        pltpu.async_copy(x_hbm.at[srcv.at[1]], r1, gs1)
        _step(0, 0, False, True)
        for j in range(1, 4):
            _step(j, j, True, True)

        def _quad(t, carry):
            for k in range(4):
                _step(4 * t + k, k, True, True)
            return carry

        lax.fori_loop(1, HCH // 4 - 1, _quad, 0)
        _step(HCH - 4, 0, True, True)
        _step(HCH - 3, 1, True, True)
        _step(HCH - 2, 2, True, False)
        _step(HCH - 1, 3, True, False)
        pltpu.make_async_copy(r3, aggsh.at[dstv.at[HCH - 1]], ss3).wait()

    plsc.subcore_barrier()
    pltpu.sync_copy(aggsh.at[pl.ds(sid * RPS, RPS)],
                    out_hbm.at[pl.ds(cid * HALF + sid * RPS, RPS)])


_sc_agg = functools.partial(
    pl.kernel,
    out_type=jax.ShapeDtypeStruct((NC * HALF, D), jnp.float32),
    mesh=plsc.VectorSubcoreMesh(core_axis_name="c", subcore_axis_name="s"),
    scratch_types=[
        pltpu.VMEM((HCH, CH), jnp.int32),
        pltpu.VMEM((HCH, CH), jnp.int32),
        pltpu.VMEM((CH, D), jnp.float32),
        pltpu.VMEM((CH, D), jnp.float32),
        pltpu.VMEM((CH, D), jnp.float32),
        pltpu.VMEM((CH, D), jnp.float32),
        pltpu.VMEM((8, D), jnp.float32),
        pltpu.VMEM_SHARED((AROWS, D), jnp.float32),
        pltpu.SemaphoreType.DMA,
        pltpu.SemaphoreType.DMA,
        pltpu.SemaphoreType.DMA,
        pltpu.SemaphoreType.DMA,
        pltpu.SemaphoreType.DMA,
        pltpu.SemaphoreType.DMA,
        pltpu.SemaphoreType.DMA,
        pltpu.SemaphoreType.DMA,
    ],
    name="sc_edge_segment_sum",
)(_sc_agg_body)


def _mlp_body(x_ref, agg_ref, w1a_ref, w1b_ref, b1_ref,
              gamma_ref, beta_ref, w2_ref, b2_ref, out_ref):
    h = jnp.dot(x_ref[...], w1a_ref[...], preferred_element_type=jnp.float32)
    h = h + jnp.dot(agg_ref[:N_NODES], w1b_ref[...],
                    preferred_element_type=jnp.float32)
    h = jnp.maximum(h + b1_ref[...], 0.0)
    mean = jnp.mean(h, axis=0, keepdims=True)
    cen = h - mean
    var = jnp.mean(cen * cen, axis=0, keepdims=True)
    hn = cen * (lax.rsqrt(var + 1e-5) * gamma_ref[...]) + beta_ref[...]
    out_ref[...] = (
        jnp.dot(hn, w2_ref[...], preferred_element_type=jnp.float32)
        + b2_ref[...])


def kernel(x, edge_index, W1, b1, gamma, beta, W2, b2):
    npad = EPAD - N_EDGES
    src = jnp.concatenate(
        [edge_index[0], jnp.zeros((npad,), jnp.int32)]
    ).reshape(NS, 2, HCH, CH)
    dst = jnp.concatenate(
        [edge_index[1], jnp.full((npad,), N_NODES, jnp.int32)]
    ).reshape(NS, 2, HCH, CH)
    agg = _sc_agg(src, dst, x)
    return pl.pallas_call(
        _mlp_body,
        out_shape=jax.ShapeDtypeStruct((N_NODES, D), jnp.float32),
    )(x, agg, W1[:D], W1[D:], b1.reshape(1, D),
      gamma.reshape(1, D), beta.reshape(1, D), W2, b2.reshape(1, D))


# speedup vs baseline: 1.5684x; 1.5684x over previous
import functools

import jax
import jax.numpy as jnp
from jax import lax
from jax.experimental import pallas as pl
from jax.experimental.pallas import tpu as pltpu
from jax.experimental.pallas import tpu_sc as plsc

N_NODES = 10000
N_EDGES = 320000
D = 128

NC = 2
NS = 16
NW = NC * NS
NROW = 10240
AROWS = NROW + 8
CH = 128
HCH = 40
NCH = 2 * HCH
EPW = NCH * CH
EPAD = NW * EPW
RPS = NROW // NS


def _sc_agg_body(src_hbm, dst_hbm, x_hbm, out_hbm,
                 srcv, dstv, rows_a, rows_b, zbuf, aggsh, sem_a, sem_b):
    cid = lax.axis_index("c")
    sid = lax.axis_index("s")
    wid = cid * NS + sid

    for r in range(8):
        for c in range(D // 16):
            zbuf[r, pl.ds(c * 16, 16)] = jnp.zeros((16,), jnp.float32)

    def _zfill(z, carry):
        pltpu.sync_copy(zbuf, aggsh.at[pl.ds(sid * RPS + z * 8, 8)])
        return carry
    lax.fori_loop(0, RPS // 8, _zfill, 0)

    @pl.when(sid == NS - 1)
    def _zero_trash():
        pltpu.sync_copy(zbuf, aggsh.at[pl.ds(NROW, 8)])

    plsc.subcore_barrier()

    for h in range(2):
        pltpu.sync_copy(src_hbm.at[wid, h], srcv)
        pltpu.sync_copy(dst_hbm.at[wid, h], dstv)

        pltpu.async_copy(x_hbm.at[srcv.at[0]], rows_a, sem_a)

        def _pair(p, carry):
            i = 2 * p
            pltpu.async_copy(x_hbm.at[srcv.at[i + 1]], rows_b, sem_b)
            pltpu.make_async_copy(
                x_hbm.at[srcv.at[i]], rows_a, sem_a).wait()
            pltpu.sync_copy(rows_a, aggsh.at[dstv.at[i]], add=True)
            pltpu.async_copy(x_hbm.at[srcv.at[i + 2]], rows_a, sem_a)
            pltpu.make_async_copy(
                x_hbm.at[srcv.at[i + 1]], rows_b, sem_b).wait()
            pltpu.sync_copy(rows_b, aggsh.at[dstv.at[i + 1]], add=True)
            return carry

        lax.fori_loop(0, HCH // 2 - 1, _pair, 0)
        pltpu.async_copy(x_hbm.at[srcv.at[HCH - 1]], rows_b, sem_b)
        pltpu.make_async_copy(
            x_hbm.at[srcv.at[HCH - 2]], rows_a, sem_a).wait()
        pltpu.sync_copy(rows_a, aggsh.at[dstv.at[HCH - 2]], add=True)
        pltpu.make_async_copy(
            x_hbm.at[srcv.at[HCH - 1]], rows_b, sem_b).wait()
        pltpu.sync_copy(rows_b, aggsh.at[dstv.at[HCH - 1]], add=True)

    plsc.subcore_barrier()
    pltpu.sync_copy(aggsh.at[pl.ds(sid * RPS, RPS)],
                    out_hbm.at[pl.ds(cid * NROW + sid * RPS, RPS)])


_sc_agg = functools.partial(
    pl.kernel,
    out_type=jax.ShapeDtypeStruct((NC * NROW, D), jnp.float32),
    mesh=plsc.VectorSubcoreMesh(core_axis_name="c", subcore_axis_name="s"),
    scratch_types=[
        pltpu.VMEM((HCH, CH), jnp.int32),
        pltpu.VMEM((HCH, CH), jnp.int32),
        pltpu.VMEM((CH, D), jnp.float32),
        pltpu.VMEM((CH, D), jnp.float32),
        pltpu.VMEM((8, D), jnp.float32),
        pltpu.VMEM_SHARED((AROWS, D), jnp.float32),
        pltpu.SemaphoreType.DMA,
        pltpu.SemaphoreType.DMA,
    ],
    name="sc_edge_segment_sum",
)(_sc_agg_body)


def _mlp_body(x_ref, p0_ref, p1_ref, w1a_ref, w1b_ref, b1_ref,
              gamma_ref, beta_ref, w2_ref, b2_ref, out_ref):
    agg = p0_ref[:N_NODES] + p1_ref[:N_NODES]
    h = jnp.dot(x_ref[...], w1a_ref[...], preferred_element_type=jnp.float32)
    h = h + jnp.dot(agg, w1b_ref[...], preferred_element_type=jnp.float32)
    h = jnp.maximum(h + b1_ref[...], 0.0)
    mean = jnp.mean(h, axis=0, keepdims=True)
    cen = h - mean
    var = jnp.mean(cen * cen, axis=0, keepdims=True)
    hn = cen * (lax.rsqrt(var + 1e-5) * gamma_ref[...]) + beta_ref[...]
    out_ref[...] = (
        jnp.dot(hn, w2_ref[...], preferred_element_type=jnp.float32)
        + b2_ref[...])


def kernel(x, edge_index, W1, b1, gamma, beta, W2, b2):
    npad = EPAD - N_EDGES
    src = jnp.concatenate(
        [edge_index[0], jnp.zeros((npad,), jnp.int32)]
    ).reshape(NW, 2, HCH, CH)
    dst = jnp.concatenate(
        [edge_index[1], jnp.full((npad,), NROW, jnp.int32)]
    ).reshape(NW, 2, HCH, CH)
    partials = _sc_agg(src, dst, x)
    p = partials.reshape(NC, NROW, D)
    return pl.pallas_call(
        _mlp_body,
        out_shape=jax.ShapeDtypeStruct((N_NODES, D), jnp.float32),
    )(x, p[0], p[1], W1[:D], W1[D:], b1.reshape(1, D),
      gamma.reshape(1, D), beta.reshape(1, D), W2, b2.reshape(1, D))


# speedup vs baseline: 1.5798x; 1.0073x over previous
import functools

import jax
import jax.numpy as jnp
from jax import lax
from jax.experimental import pallas as pl
from jax.experimental.pallas import tpu as pltpu
from jax.experimental.pallas import tpu_sc as plsc

N_NODES = 10000
N_EDGES = 320000
D = 128

NC = 2
NS = 16
NW = NC * NS
NROW = 10240
AROWS = NROW + 8
CH = 128
HCH = 40
NCH = 2 * HCH
EPW = NCH * CH
EPAD = NW * EPW
RPS = NROW // NS


def _sc_agg_body(src_hbm, dst_hbm, x_hbm, out_hbm,
                 srcv, dstv, rows_a, rows_b, aggsh, sem_a, sem_b):
    cid = lax.axis_index("c")
    sid = lax.axis_index("s")
    wid = cid * NS + sid

    def _zrow(r, carry):
        for c in range(D // 16):
            rows_a[r, pl.ds(c * 16, 16)] = jnp.zeros((16,), jnp.float32)
        return carry
    lax.fori_loop(0, CH, _zrow, 0)
    for z in range(RPS // CH):
        pltpu.sync_copy(rows_a, aggsh.at[pl.ds(sid * RPS + z * CH, CH)])

    @pl.when(sid == NS - 1)
    def _zero_trash():
        pltpu.sync_copy(rows_a.at[pl.ds(0, 8)], aggsh.at[pl.ds(NROW, 8)])

    plsc.subcore_barrier()

    for h in range(2):
        pltpu.sync_copy(src_hbm.at[wid, h], srcv)
        pltpu.sync_copy(dst_hbm.at[wid, h], dstv)

        pltpu.async_copy(x_hbm.at[srcv.at[0]], rows_a, sem_a)

        def _pair(p, carry):
            i = 2 * p
            pltpu.async_copy(x_hbm.at[srcv.at[i + 1]], rows_b, sem_b)
            pltpu.make_async_copy(
                x_hbm.at[srcv.at[i]], rows_a, sem_a).wait()
            pltpu.sync_copy(rows_a, aggsh.at[dstv.at[i]], add=True)
            pltpu.async_copy(x_hbm.at[srcv.at[i + 2]], rows_a, sem_a)
            pltpu.make_async_copy(
                x_hbm.at[srcv.at[i + 1]], rows_b, sem_b).wait()
            pltpu.sync_copy(rows_b, aggsh.at[dstv.at[i + 1]], add=True)
            return carry

        lax.fori_loop(0, HCH // 2 - 1, _pair, 0)
        pltpu.async_copy(x_hbm.at[srcv.at[HCH - 1]], rows_b, sem_b)
        pltpu.make_async_copy(
            x_hbm.at[srcv.at[HCH - 2]], rows_a, sem_a).wait()
        pltpu.sync_copy(rows_a, aggsh.at[dstv.at[HCH - 2]], add=True)
        pltpu.make_async_copy(
            x_hbm.at[srcv.at[HCH - 1]], rows_b, sem_b).wait()
        pltpu.sync_copy(rows_b, aggsh.at[dstv.at[HCH - 1]], add=True)

    plsc.subcore_barrier()
    pltpu.sync_copy(aggsh.at[pl.ds(sid * RPS, RPS)],
                    out_hbm.at[pl.ds(cid * NROW + sid * RPS, RPS)])


_sc_agg = functools.partial(
    pl.kernel,
    out_type=jax.ShapeDtypeStruct((NC * NROW, D), jnp.float32),
    mesh=plsc.VectorSubcoreMesh(core_axis_name="c", subcore_axis_name="s"),
    scratch_types=[
        pltpu.VMEM((HCH, CH), jnp.int32),
        pltpu.VMEM((HCH, CH), jnp.int32),
        pltpu.VMEM((CH, D), jnp.float32),
        pltpu.VMEM((CH, D), jnp.float32),
        pltpu.VMEM_SHARED((AROWS, D), jnp.float32),
        pltpu.SemaphoreType.DMA,
        pltpu.SemaphoreType.DMA,
    ],
    name="sc_edge_segment_sum",
)(_sc_agg_body)


def _mlp_body(x_ref, p0_ref, p1_ref, w1a_ref, w1b_ref, b1_ref,
              gamma_ref, beta_ref, w2_ref, b2_ref, out_ref):
    agg = p0_ref[:N_NODES] + p1_ref[:N_NODES]
    h = jnp.dot(x_ref[...], w1a_ref[...], preferred_element_type=jnp.float32)
    h = h + jnp.dot(agg, w1b_ref[...], preferred_element_type=jnp.float32)
    h = jnp.maximum(h + b1_ref[...], 0.0)
    mean = jnp.mean(h, axis=0, keepdims=True)
    cen = h - mean
    var = jnp.mean(cen * cen, axis=0, keepdims=True)
    hn = cen * (lax.rsqrt(var + 1e-5) * gamma_ref[...]) + beta_ref[...]
    out_ref[...] = (
        jnp.dot(hn, w2_ref[...], preferred_element_type=jnp.float32)
        + b2_ref[...])


def kernel(x, edge_index, W1, b1, gamma, beta, W2, b2):
    npad = EPAD - N_EDGES
    src = jnp.concatenate(
        [edge_index[0], jnp.zeros((npad,), jnp.int32)]
    ).reshape(NW, 2, HCH, CH)
    dst = jnp.concatenate(
        [edge_index[1], jnp.full((npad,), NROW, jnp.int32)]
    ).reshape(NW, 2, HCH, CH)
    partials = _sc_agg(src, dst, x)
    p = partials.reshape(NC, NROW, D)
    return pl.pallas_call(
        _mlp_body,
        out_shape=jax.ShapeDtypeStruct((N_NODES, D), jnp.float32),
    )(x, p[0], p[1], W1[:D], W1[D:], b1.reshape(1, D),
      gamma.reshape(1, D), beta.reshape(1, D), W2, b2.reshape(1, D))


# speedup vs baseline: 1.5800x; 1.0001x over previous
import functools

import jax
import jax.numpy as jnp
from jax import lax
from jax.experimental import pallas as pl
from jax.experimental.pallas import tpu as pltpu
from jax.experimental.pallas import tpu_sc as plsc

N_NODES = 10000
N_EDGES = 320000
D = 128

NC = 2
NS = 16
NW = NC * NS
NROW = 10240
AROWS = NROW + 8
CH = 128
HCH = 40
NCH = 2 * HCH
EPW = NCH * CH
EPAD = NW * EPW
RPS = NROW // NS


def _sc_agg_body(src_hbm, dst_hbm, x_hbm, out_hbm,
                 srcv, dstv, rows_a, rows_b, aggsh, sem_a, sem_b):
    cid = lax.axis_index("c")
    sid = lax.axis_index("s")
    wid = cid * NS + sid

    def _zrow(r, carry):
        for c in range(D // 16):
            rows_a[r, pl.ds(c * 16, 16)] = jnp.zeros((16,), jnp.float32)
        return carry
    lax.fori_loop(0, CH, _zrow, 0)
    for z in range(RPS // CH):
        pltpu.sync_copy(rows_a, aggsh.at[pl.ds(sid * RPS + z * CH, CH)])

    @pl.when(sid == NS - 1)
    def _zero_trash():
        pltpu.sync_copy(rows_a.at[pl.ds(0, 8)], aggsh.at[pl.ds(NROW, 8)])

    plsc.subcore_barrier()

    for h in range(2):
        pltpu.sync_copy(src_hbm.at[wid, h], srcv)
        pltpu.sync_copy(dst_hbm.at[wid, h], dstv)

        pltpu.async_copy(x_hbm.at[srcv.at[0]], rows_a, sem_a)

        def _pair(p, carry):
            i = 2 * p
            pltpu.async_copy(x_hbm.at[srcv.at[i + 1]], rows_b, sem_b)
            pltpu.make_async_copy(
                x_hbm.at[srcv.at[i]], rows_a, sem_a).wait()
            pltpu.sync_copy(rows_a, aggsh.at[dstv.at[i]], add=True)
            pltpu.async_copy(x_hbm.at[srcv.at[i + 2]], rows_a, sem_a)
            pltpu.make_async_copy(
                x_hbm.at[srcv.at[i + 1]], rows_b, sem_b).wait()
            pltpu.sync_copy(rows_b, aggsh.at[dstv.at[i + 1]], add=True)
            return carry

        lax.fori_loop(0, HCH // 2 - 1, _pair, 0)
        pltpu.async_copy(x_hbm.at[srcv.at[HCH - 1]], rows_b, sem_b)
        pltpu.make_async_copy(
            x_hbm.at[srcv.at[HCH - 2]], rows_a, sem_a).wait()
        pltpu.sync_copy(rows_a, aggsh.at[dstv.at[HCH - 2]], add=True)
        pltpu.make_async_copy(
            x_hbm.at[srcv.at[HCH - 1]], rows_b, sem_b).wait()
        pltpu.sync_copy(rows_b, aggsh.at[dstv.at[HCH - 1]], add=True)

    plsc.subcore_barrier()
    pltpu.sync_copy(aggsh.at[pl.ds(sid * RPS, RPS)],
                    out_hbm.at[pl.ds(cid * NROW + sid * RPS, RPS)])


_sc_agg = functools.partial(
    pl.kernel,
    out_type=jax.ShapeDtypeStruct((NC * NROW, D), jnp.float32),
    mesh=plsc.VectorSubcoreMesh(core_axis_name="c", subcore_axis_name="s"),
    scratch_types=[
        pltpu.VMEM((HCH, CH), jnp.int32),
        pltpu.VMEM((HCH, CH), jnp.int32),
        pltpu.VMEM((CH, D), jnp.float32),
        pltpu.VMEM((CH, D), jnp.float32),
        pltpu.VMEM_SHARED((AROWS, D), jnp.float32),
        pltpu.SemaphoreType.DMA,
        pltpu.SemaphoreType.DMA,
    ],
    name="sc_edge_segment_sum",
)(_sc_agg_body)


def _mlp_body(x_ref, p0_ref, p1_ref, w1a_ref, w1b_ref, b1_ref,
              gamma_ref, beta_ref, w2_ref, b2_ref, out_ref):
    agg = p0_ref[:N_NODES] + p1_ref[:N_NODES]
    h = jnp.dot(x_ref[...], w1a_ref[...], preferred_element_type=jnp.float32)
    h = h + jnp.dot(agg, w1b_ref[...], preferred_element_type=jnp.float32)
    h = jnp.maximum(h + b1_ref[...], 0.0)
    mean = jnp.mean(h, axis=0, keepdims=True)
    cen = h - mean
    var = jnp.mean(cen * cen, axis=0, keepdims=True)
    hn = cen * (lax.rsqrt(var + 1e-5) * gamma_ref[...]) + beta_ref[...]
    out_ref[...] = (
        jnp.dot(hn, w2_ref[...], preferred_element_type=jnp.float32)
        + b2_ref[...])


def kernel(x, edge_index, W1, b1, gamma, beta, W2, b2):
    npad = EPAD - N_EDGES
    src = jnp.concatenate(
        [edge_index[0], jnp.zeros((npad,), jnp.int32)]
    ).reshape(NW, 2, HCH, CH)
    pad_dst = N_NODES + jnp.arange(npad, dtype=jnp.int32) % (NROW - N_NODES)
    dst = jnp.concatenate(
        [edge_index[1], pad_dst]
    ).reshape(NW, 2, HCH, CH)
    partials = _sc_agg(src, dst, x)
    p = partials.reshape(NC, NROW, D)
    return pl.pallas_call(
        _mlp_body,
        out_shape=jax.ShapeDtypeStruct((N_NODES, D), jnp.float32),
    )(x, p[0], p[1], W1[:D], W1[D:], b1.reshape(1, D),
      gamma.reshape(1, D), beta.reshape(1, D), W2, b2.reshape(1, D))


# speedup vs baseline: 5.1854x; 3.2819x over previous
import functools

import jax
import jax.numpy as jnp
from jax import lax
from jax.experimental import pallas as pl
from jax.experimental.pallas import tpu as pltpu
from jax.experimental.pallas import tpu_sc as plsc

N_NODES = 10000
N_EDGES = 320000
D = 128

NC = 2
NS = 16
NW = NC * NS
NROW = 10240
AROWS = NROW + 8
CH = 128
HCH = 40
NCH = 2 * HCH
EPW = NCH * CH
EPAD = NW * EPW
RPS = NROW // NS


def _sc_agg_body(src_hbm, dst_hbm, x_hbm, out_hbm,
                 srcv, dstv, rows_a, rows_b, aggsh, sem_a, sem_b):
    cid = lax.axis_index("c")
    sid = lax.axis_index("s")
    wid = cid * NS + sid

    def _zrow(r, carry):
        for c in range(D // 16):
            rows_a[r, pl.ds(c * 16, 16)] = jnp.zeros((16,), jnp.float32)
        return carry
    lax.fori_loop(0, CH, _zrow, 0)
    for z in range(RPS // CH):
        pltpu.sync_copy(rows_a, aggsh.at[pl.ds(sid * RPS + z * CH, CH)])

    @pl.when(sid == NS - 1)
    def _zero_trash():
        pltpu.sync_copy(rows_a.at[pl.ds(0, 8)], aggsh.at[pl.ds(NROW, 8)])

    plsc.subcore_barrier()

    for h in range(2):
        pltpu.sync_copy(src_hbm.at[wid, h], srcv)
        pltpu.sync_copy(dst_hbm.at[wid, h], dstv)

        pltpu.async_copy(x_hbm.at[srcv.at[0]], rows_a, sem_a)

        def _pair(p, carry):
            i = 2 * p
            pltpu.async_copy(x_hbm.at[srcv.at[i + 1]], rows_b, sem_b)
            pltpu.make_async_copy(
                x_hbm.at[srcv.at[i]], rows_a, sem_a).wait()
            pltpu.sync_copy(rows_a, aggsh.at[dstv.at[i]], add=True)
            pltpu.async_copy(x_hbm.at[srcv.at[i + 2]], rows_a, sem_a)
            pltpu.make_async_copy(
                x_hbm.at[srcv.at[i + 1]], rows_b, sem_b).wait()
            pltpu.sync_copy(rows_b, aggsh.at[dstv.at[i + 1]], add=True)
            return carry

        lax.fori_loop(0, HCH // 2 - 1, _pair, 0)
        pltpu.async_copy(x_hbm.at[srcv.at[HCH - 1]], rows_b, sem_b)
        pltpu.make_async_copy(
            x_hbm.at[srcv.at[HCH - 2]], rows_a, sem_a).wait()
        pltpu.sync_copy(rows_a, aggsh.at[dstv.at[HCH - 2]], add=True)
        pltpu.make_async_copy(
            x_hbm.at[srcv.at[HCH - 1]], rows_b, sem_b).wait()
        pltpu.sync_copy(rows_b, aggsh.at[dstv.at[HCH - 1]], add=True)

    plsc.subcore_barrier()
    pltpu.sync_copy(aggsh.at[pl.ds(sid * RPS, RPS)],
                    out_hbm.at[pl.ds(cid * NROW + sid * RPS, RPS)])


_sc_agg = functools.partial(
    pl.kernel,
    out_type=jax.ShapeDtypeStruct((NC * NROW, D), jnp.float32),
    mesh=plsc.VectorSubcoreMesh(core_axis_name="c", subcore_axis_name="s"),
    scratch_types=[
        pltpu.VMEM((HCH, CH), jnp.int32),
        pltpu.VMEM((HCH, CH), jnp.int32),
        pltpu.VMEM((CH, D), jnp.float32),
        pltpu.VMEM((CH, D), jnp.float32),
        pltpu.VMEM_SHARED((AROWS, D), jnp.float32),
        pltpu.SemaphoreType.DMA,
        pltpu.SemaphoreType.DMA,
    ],
    name="sc_edge_segment_sum",
)(_sc_agg_body)


def _mlp_body(x_ref, p0_ref, p1_ref, w1a_ref, w1b_ref, b1_ref,
              gamma_ref, beta_ref, w2_ref, b2_ref, out_ref):
    agg = p0_ref[:N_NODES] + p1_ref[:N_NODES]
    h = jnp.dot(x_ref[...], w1a_ref[...], preferred_element_type=jnp.float32)
    h = h + jnp.dot(agg, w1b_ref[...], preferred_element_type=jnp.float32)
    h = jnp.maximum(h + b1_ref[...], 0.0)
    mean = jnp.mean(h, axis=0, keepdims=True)
    cen = h - mean
    var = jnp.mean(cen * cen, axis=0, keepdims=True)
    hn = cen * (lax.rsqrt(var + 1e-5) * gamma_ref[...]) + beta_ref[...]
    out_ref[...] = (
        jnp.dot(hn, w2_ref[...], preferred_element_type=jnp.float32)
        + b2_ref[...])


def kernel(x, edge_index, W1, b1, gamma, beta, W2, b2):
    npad = EPAD - N_EDGES
    pad_src = jnp.arange(npad, dtype=jnp.int32) % N_NODES
    src = jnp.concatenate(
        [edge_index[0], pad_src]
    ).reshape(NW, 2, HCH, CH)
    pad_dst = N_NODES + jnp.arange(npad, dtype=jnp.int32) % (NROW - N_NODES)
    dst = jnp.concatenate(
        [edge_index[1], pad_dst]
    ).reshape(NW, 2, HCH, CH)
    partials = _sc_agg(src, dst, x)
    p = partials.reshape(NC, NROW, D)
    return pl.pallas_call(
        _mlp_body,
        out_shape=jax.ShapeDtypeStruct((N_NODES, D), jnp.float32),
    )(x, p[0], p[1], W1[:D], W1[D:], b1.reshape(1, D),
      gamma.reshape(1, D), beta.reshape(1, D), W2, b2.reshape(1, D))


# speedup vs baseline: 5.2727x; 1.0168x over previous
import functools

import jax
import jax.numpy as jnp
from jax import lax
from jax.experimental import pallas as pl
from jax.experimental.pallas import tpu as pltpu
from jax.experimental.pallas import tpu_sc as plsc

N_NODES = 10000
N_EDGES = 320000
D = 128

NC = 2
NS = 16
NW = NC * NS
NROW = 10240
CH = 100
HCH = 50
NCH = 2 * HCH
EPW = NCH * CH
RPS = NROW // NS


def _sc_agg_body(src_hbm, dst_hbm, x_hbm, out_hbm,
                 srcv, dstv, rows_a, rows_b, aggsh, sem_a, sem_b):
    cid = lax.axis_index("c")
    sid = lax.axis_index("s")
    wid = cid * NS + sid

    def _zrow(r, carry):
        for c in range(D // 16):
            rows_a[r, pl.ds(c * 16, 16)] = jnp.zeros((16,), jnp.float32)
        return carry
    lax.fori_loop(0, 80, _zrow, 0)
    for z in range(RPS // 80):
        pltpu.sync_copy(rows_a.at[pl.ds(0, 80)],
                        aggsh.at[pl.ds(sid * RPS + z * 80, 80)])

    plsc.subcore_barrier()

    for h in range(2):
        pltpu.sync_copy(src_hbm.at[wid, h], srcv)
        pltpu.sync_copy(dst_hbm.at[wid, h], dstv)

        pltpu.async_copy(x_hbm.at[srcv.at[0]], rows_a, sem_a)

        def _pair(p, carry):
            i = 2 * p
            pltpu.async_copy(x_hbm.at[srcv.at[i + 1]], rows_b, sem_b)
            pltpu.make_async_copy(
                x_hbm.at[srcv.at[i]], rows_a, sem_a).wait()
            pltpu.sync_copy(rows_a, aggsh.at[dstv.at[i]], add=True)
            pltpu.async_copy(x_hbm.at[srcv.at[i + 2]], rows_a, sem_a)
            pltpu.make_async_copy(
                x_hbm.at[srcv.at[i + 1]], rows_b, sem_b).wait()
            pltpu.sync_copy(rows_b, aggsh.at[dstv.at[i + 1]], add=True)
            return carry

        lax.fori_loop(0, HCH // 2 - 1, _pair, 0)
        pltpu.async_copy(x_hbm.at[srcv.at[HCH - 1]], rows_b, sem_b)
        pltpu.make_async_copy(
            x_hbm.at[srcv.at[HCH - 2]], rows_a, sem_a).wait()
        pltpu.sync_copy(rows_a, aggsh.at[dstv.at[HCH - 2]], add=True)
        pltpu.make_async_copy(
            x_hbm.at[srcv.at[HCH - 1]], rows_b, sem_b).wait()
        pltpu.sync_copy(rows_b, aggsh.at[dstv.at[HCH - 1]], add=True)

    plsc.subcore_barrier()
    pltpu.sync_copy(aggsh.at[pl.ds(sid * RPS, RPS)],
                    out_hbm.at[pl.ds(cid * NROW + sid * RPS, RPS)])


_sc_agg = functools.partial(
    pl.kernel,
    out_type=jax.ShapeDtypeStruct((NC * NROW, D), jnp.float32),
    mesh=plsc.VectorSubcoreMesh(core_axis_name="c", subcore_axis_name="s"),
    scratch_types=[
        pltpu.VMEM((HCH, CH), jnp.int32),
        pltpu.VMEM((HCH, CH), jnp.int32),
        pltpu.VMEM((CH, D), jnp.float32),
        pltpu.VMEM((CH, D), jnp.float32),
        pltpu.VMEM_SHARED((NROW, D), jnp.float32),
        pltpu.SemaphoreType.DMA,
        pltpu.SemaphoreType.DMA,
    ],
    name="sc_edge_segment_sum",
)(_sc_agg_body)


def _mlp_body(x_ref, part_ref, w1a_ref, w1b_ref, b1_ref,
              gamma_ref, beta_ref, w2_ref, b2_ref, out_ref):
    agg = part_ref[:N_NODES] + part_ref[NROW:NROW + N_NODES]
    h = jnp.dot(x_ref[...], w1a_ref[...], preferred_element_type=jnp.float32)
    h = h + jnp.dot(agg, w1b_ref[...], preferred_element_type=jnp.float32)
    h = jnp.maximum(h + b1_ref[...], 0.0)
    mean = jnp.mean(h, axis=0, keepdims=True)
    cen = h - mean
    var = jnp.mean(cen * cen, axis=0, keepdims=True)
    hn = cen * (lax.rsqrt(var + 1e-5) * gamma_ref[...]) + beta_ref[...]
    out_ref[...] = (
        jnp.dot(hn, w2_ref[...], preferred_element_type=jnp.float32)
        + b2_ref[...])


def kernel(x, edge_index, W1, b1, gamma, beta, W2, b2):
    src = edge_index[0].reshape(NW, 2, HCH, CH)
    dst = edge_index[1].reshape(NW, 2, HCH, CH)
    partials = _sc_agg(src, dst, x)
    return pl.pallas_call(
        _mlp_body,
        out_shape=jax.ShapeDtypeStruct((N_NODES, D), jnp.float32),
    )(x, partials, W1[:D], W1[D:], b1.reshape(1, D),
      gamma.reshape(1, D), beta.reshape(1, D), W2, b2.reshape(1, D))


# speedup vs baseline: 5.6749x; 1.0763x over previous
import functools

import jax
import jax.numpy as jnp
from jax import lax
from jax.experimental import pallas as pl
from jax.experimental.pallas import tpu as pltpu
from jax.experimental.pallas import tpu_sc as plsc

N_NODES = 10000
N_EDGES = 320000
D = 128

NC = 2
NS = 16
NW = NC * NS
NROW = 10240
CH = 100
HCH = 50
NCH = 2 * HCH
EPW = NCH * CH
RPS = NROW // NS


def _sc_agg_body(edge_hbm, x_hbm, out_hbm,
                 srcv, dstv, rows_a, rows_b, aggsh, sem_a, sem_b):
    cid = lax.axis_index("c")
    sid = lax.axis_index("s")
    wid = cid * NS + sid

    def _zrow(r, carry):
        for c in range(D // 16):
            rows_a[r, pl.ds(c * 16, 16)] = jnp.zeros((16,), jnp.float32)
        return carry
    lax.fori_loop(0, 80, _zrow, 0)
    for z in range(RPS // 80):
        pltpu.sync_copy(rows_a.at[pl.ds(0, 80)],
                        aggsh.at[pl.ds(sid * RPS + z * 80, 80)])

    plsc.subcore_barrier()

    for h in range(2):
        pltpu.sync_copy(edge_hbm.at[0, wid, h], srcv)
        pltpu.sync_copy(edge_hbm.at[1, wid, h], dstv)

        pltpu.async_copy(x_hbm.at[srcv.at[0]], rows_a, sem_a)

        def _pair(p, carry):
            i = 2 * p
            pltpu.async_copy(x_hbm.at[srcv.at[i + 1]], rows_b, sem_b)
            pltpu.make_async_copy(
                x_hbm.at[srcv.at[i]], rows_a, sem_a).wait()
            pltpu.sync_copy(rows_a, aggsh.at[dstv.at[i]], add=True)
            pltpu.async_copy(x_hbm.at[srcv.at[i + 2]], rows_a, sem_a)
            pltpu.make_async_copy(
                x_hbm.at[srcv.at[i + 1]], rows_b, sem_b).wait()
            pltpu.sync_copy(rows_b, aggsh.at[dstv.at[i + 1]], add=True)
            return carry

        lax.fori_loop(0, HCH // 2 - 1, _pair, 0)
        pltpu.async_copy(x_hbm.at[srcv.at[HCH - 1]], rows_b, sem_b)
        pltpu.make_async_copy(
            x_hbm.at[srcv.at[HCH - 2]], rows_a, sem_a).wait()
        pltpu.sync_copy(rows_a, aggsh.at[dstv.at[HCH - 2]], add=True)
        pltpu.make_async_copy(
            x_hbm.at[srcv.at[HCH - 1]], rows_b, sem_b).wait()
        pltpu.sync_copy(rows_b, aggsh.at[dstv.at[HCH - 1]], add=True)

    plsc.subcore_barrier()
    pltpu.sync_copy(aggsh.at[pl.ds(sid * RPS, RPS)],
                    out_hbm.at[pl.ds(cid * NROW + sid * RPS, RPS)])


_sc_agg = functools.partial(
    pl.kernel,
    out_type=jax.ShapeDtypeStruct((NC * NROW, D), jnp.float32),
    mesh=plsc.VectorSubcoreMesh(core_axis_name="c", subcore_axis_name="s"),
    scratch_types=[
        pltpu.VMEM((HCH, CH), jnp.int32),
        pltpu.VMEM((HCH, CH), jnp.int32),
        pltpu.VMEM((CH, D), jnp.float32),
        pltpu.VMEM((CH, D), jnp.float32),
        pltpu.VMEM_SHARED((NROW, D), jnp.float32),
        pltpu.SemaphoreType.DMA,
        pltpu.SemaphoreType.DMA,
    ],
    name="sc_edge_segment_sum",
)(_sc_agg_body)


def _mlp_body(x_ref, part_ref, w1a_ref, w1b_ref, b1_ref,
              gamma_ref, beta_ref, w2_ref, b2_ref, out_ref):
    agg = part_ref[:N_NODES] + part_ref[NROW:NROW + N_NODES]
    h = jnp.dot(x_ref[...], w1a_ref[...], preferred_element_type=jnp.float32)
    h = h + jnp.dot(agg, w1b_ref[...], preferred_element_type=jnp.float32)
    h = jnp.maximum(h + b1_ref[...], 0.0)
    mean = jnp.mean(h, axis=0, keepdims=True)
    cen = h - mean
    var = jnp.mean(cen * cen, axis=0, keepdims=True)
    hn = cen * (lax.rsqrt(var + 1e-5) * gamma_ref[...]) + beta_ref[...]
    out_ref[...] = (
        jnp.dot(hn, w2_ref[...], preferred_element_type=jnp.float32)
        + b2_ref[...])


def kernel(x, edge_index, W1, b1, gamma, beta, W2, b2):
    edges = edge_index.reshape(2, NW, 2, HCH, CH)
    partials = _sc_agg(edges, x)
    return pl.pallas_call(
        _mlp_body,
        out_shape=jax.ShapeDtypeStruct((N_NODES, D), jnp.float32),
    )(x, partials, W1[:D], W1[D:], b1.reshape(1, D),
      gamma.reshape(1, D), beta.reshape(1, D), W2, b2.reshape(1, D))


# speedup vs baseline: 5.7905x; 1.0204x over previous
import functools

import jax
import jax.numpy as jnp
from jax import lax
from jax.experimental import pallas as pl
from jax.experimental.pallas import tpu as pltpu
from jax.experimental.pallas import tpu_sc as plsc

N_NODES = 10000
N_EDGES = 320000
D = 128

NC = 2
NS = 16
NW = NC * NS
NROW = 10240
CH = 125
HCH = 40
NCH = 2 * HCH
EPW = NCH * CH
RPS = NROW // NS


def _sc_agg_body(edge_hbm, x_hbm, out_hbm,
                 srcv, dstv, rows_a, rows_b, aggsh, sem_a, sem_b):
    cid = lax.axis_index("c")
    sid = lax.axis_index("s")
    wid = cid * NS + sid

    def _zrow(r, carry):
        for c in range(D // 16):
            rows_a[r, pl.ds(c * 16, 16)] = jnp.zeros((16,), jnp.float32)
        return carry
    lax.fori_loop(0, 80, _zrow, 0)
    for z in range(RPS // 80):
        pltpu.sync_copy(rows_a.at[pl.ds(0, 80)],
                        aggsh.at[pl.ds(sid * RPS + z * 80, 80)])

    plsc.subcore_barrier()

    for h in range(2):
        pltpu.sync_copy(edge_hbm.at[0, wid, h], srcv)
        pltpu.sync_copy(edge_hbm.at[1, wid, h], dstv)

        pltpu.async_copy(x_hbm.at[srcv.at[0]], rows_a, sem_a)

        def _pair(p, carry):
            i = 2 * p
            pltpu.async_copy(x_hbm.at[srcv.at[i + 1]], rows_b, sem_b)
            pltpu.make_async_copy(
                x_hbm.at[srcv.at[i]], rows_a, sem_a).wait()
            pltpu.sync_copy(rows_a, aggsh.at[dstv.at[i]], add=True)
            pltpu.async_copy(x_hbm.at[srcv.at[i + 2]], rows_a, sem_a)
            pltpu.make_async_copy(
                x_hbm.at[srcv.at[i + 1]], rows_b, sem_b).wait()
            pltpu.sync_copy(rows_b, aggsh.at[dstv.at[i + 1]], add=True)
            return carry

        lax.fori_loop(0, HCH // 2 - 1, _pair, 0)
        pltpu.async_copy(x_hbm.at[srcv.at[HCH - 1]], rows_b, sem_b)
        pltpu.make_async_copy(
            x_hbm.at[srcv.at[HCH - 2]], rows_a, sem_a).wait()
        pltpu.sync_copy(rows_a, aggsh.at[dstv.at[HCH - 2]], add=True)
        pltpu.make_async_copy(
            x_hbm.at[srcv.at[HCH - 1]], rows_b, sem_b).wait()
        pltpu.sync_copy(rows_b, aggsh.at[dstv.at[HCH - 1]], add=True)

    plsc.subcore_barrier()
    pltpu.sync_copy(aggsh.at[pl.ds(sid * RPS, RPS)],
                    out_hbm.at[pl.ds(cid * NROW + sid * RPS, RPS)])


_sc_agg = functools.partial(
    pl.kernel,
    out_type=jax.ShapeDtypeStruct((NC * NROW, D), jnp.float32),
    mesh=plsc.VectorSubcoreMesh(core_axis_name="c", subcore_axis_name="s"),
    scratch_types=[
        pltpu.VMEM((HCH, CH), jnp.int32),
        pltpu.VMEM((HCH, CH), jnp.int32),
        pltpu.VMEM((CH, D), jnp.float32),
        pltpu.VMEM((CH, D), jnp.float32),
        pltpu.VMEM_SHARED((NROW, D), jnp.float32),
        pltpu.SemaphoreType.DMA,
        pltpu.SemaphoreType.DMA,
    ],
    name="sc_edge_segment_sum",
)(_sc_agg_body)


def _hx_body(x_ref, w1a_ref, b1_ref, hx_ref):
    hx_ref[...] = (
        jnp.dot(x_ref[...], w1a_ref[...], preferred_element_type=jnp.float32)
        + b1_ref[...])


def _mlp_body(hx_ref, part_ref, w1b_ref,
              gamma_ref, beta_ref, w2_ref, b2_ref, out_ref):
    agg = part_ref[:N_NODES] + part_ref[NROW:NROW + N_NODES]
    h = hx_ref[...] + jnp.dot(agg, w1b_ref[...],
                              preferred_element_type=jnp.float32)
    h = jnp.maximum(h, 0.0)
    mean = jnp.mean(h, axis=0, keepdims=True)
    cen = h - mean
    var = jnp.mean(cen * cen, axis=0, keepdims=True)
    hn = cen * (lax.rsqrt(var + 1e-5) * gamma_ref[...]) + beta_ref[...]
    out_ref[...] = (
        jnp.dot(hn, w2_ref[...], preferred_element_type=jnp.float32)
        + b2_ref[...])


def kernel(x, edge_index, W1, b1, gamma, beta, W2, b2):
    edges = edge_index.reshape(2, NW, 2, HCH, CH)
    partials = _sc_agg(edges, x)
    hx = pl.pallas_call(
        _hx_body,
        out_shape=jax.ShapeDtypeStruct((N_NODES, D), jnp.float32),
    )(x, W1[:D], b1.reshape(1, D))
    return pl.pallas_call(
        _mlp_body,
        out_shape=jax.ShapeDtypeStruct((N_NODES, D), jnp.float32),
    )(hx, partials, W1[D:],
      gamma.reshape(1, D), beta.reshape(1, D), W2, b2.reshape(1, D))
